# Initial kernel scaffold; baseline (speedup 1.0000x reference)
import jax
import jax.numpy as jnp
from jax.experimental import pallas as pl


def kernel(attn_score_cache, key_cache, value_cache):
    raise NotImplementedError("write your pallas kernel here")


# speedup vs baseline: 1.5682x; 1.5682x over previous
import functools

import jax
import jax.numpy as jnp
from jax import lax
from jax.experimental import pallas as pl
from jax.experimental.pallas import tpu as pltpu
from jax.experimental.pallas import tpu_sc as plsc

_HH = 512
_RECENT = 512
_CACHE = _HH + _RECENT

_B, _H, _Q, _T, _D = 8, 32, 8, 4096, 64
_SEL = _T - _RECENT
_NC, _NS = 2, 16
_NW = _NC * _NS
_PAIRS = _B * _H
_PPW = _PAIRS // _NW


def _tc_body(scores_ref, hh_ref, tau_ref, m_ref):
    s = scores_ref[0]
    hh = jnp.sum(s, axis=1)
    hh_ref[0] = hh
    bits = lax.bitcast_convert_type(hh[:, :_SEL], jnp.int32)

    def step(_, carry):
        lo, hi = carry
        mid = lo + lax.div(hi - lo, 2)
        cnt = jnp.sum((bits > mid).astype(jnp.int32), axis=1, keepdims=True)
        conv = cnt < _HH
        return jnp.where(conv, lo, mid + 1), jnp.where(conv, mid, hi)

    lo0 = jnp.zeros((_H, 1), jnp.int32)
    hi0 = jnp.full((_H, 1), jnp.int32(0x7F000000))
    tau, _ = lax.fori_loop(0, 31, step, (lo0, hi0))
    c = jnp.sum((bits > tau).astype(jnp.int32), axis=1, keepdims=True)
    tau_ref[0] = jnp.broadcast_to(tau, (_H, 128))
    m_ref[0] = jnp.broadcast_to(_HH - c, (_H, 128))


def _sc_body(hh_hbm, tau_hbm, m_hbm, k_hbm, v_hbm, kout, vout, hhout,
             hh_v, tau_v, m_v, idx_v, gidx_v, rows_v, hho_v, sem):
    wid = lax.axis_index("s") * _NC + lax.axis_index("c")
    pltpu.sync_copy(tau_hbm.at[pl.ds(wid * _PPW, 16)], tau_v)
    pltpu.sync_copy(m_hbm.at[pl.ds(wid * _PPW, 16)], m_v)
    lanes = lax.broadcasted_iota(jnp.int32, (16,), 0)

    def do_pair(p, _):
        pair = wid * _PPW + p
        pltpu.sync_copy(hh_hbm.at[pair], hh_v)
        pidx = jnp.full((16,), p, jnp.int32)
        tau_b = plsc.load_gather(tau_v, [pidx])
        m_b = plsc.load_gather(m_v, [pidx])

        def step(i, carry):
            e, off = carry
            v = hh_v[pl.ds(i * 16, 16)]
            pos = i * 16 + lanes
            gt = v > tau_b
            eq = v == tau_b
            eqc = plsc.cumsum(eq.astype(jnp.int32))
            keep_eq = jnp.logical_and(eq, (e + eqc) <= m_b)
            msk = jnp.logical_or(gt, keep_eq)
            dest = off + plsc.cumsum(msk.astype(jnp.int32)) - 1
            plsc.store_scatter(idx_v, [dest], pos, mask=msk)
            return (e + plsc.all_reduce_population_count(eq),
                    off + plsc.all_reduce_population_count(msk))

        z = jnp.zeros((16,), jnp.int32)
        lax.fori_loop(0, _SEL // 16, step, (z, z))

        def recent(j, _):
            idx_v[pl.ds(_HH + j * 16, 16)] = _SEL + j * 16 + lanes
            return 0

        lax.fori_loop(0, _RECENT // 16, recent, 0)

        base = pair * _T

        def gat(j, _):
            iv = idx_v[pl.ds(j * 16, 16)]
            hho_v[pl.ds(j * 16, 16)] = plsc.load_gather(hh_v, [iv])
            gidx_v[j // 8, pl.ds((j % 8) * 16, 16)] = iv + base
            return 0

        lax.fori_loop(0, _CACHE // 16, gat, 0)
        pltpu.sync_copy(hho_v, hhout.at[pair])
        for chunk in range(8):
            pltpu.async_copy(k_hbm.at[gidx_v.at[chunk]], rows_v.at[chunk],
                             sem).wait()
        pltpu.sync_copy(rows_v, kout.at[pair])
        for chunk in range(8):
            pltpu.async_copy(v_hbm.at[gidx_v.at[chunk]], rows_v.at[chunk],
                             sem).wait()
        pltpu.sync_copy(rows_v, vout.at[pair])
        return 0

    lax.fori_loop(0, _PPW, do_pair, 0)


@functools.cache
def _make_sc_kernel():
    mesh = plsc.VectorSubcoreMesh(core_axis_name="c", subcore_axis_name="s",
                                  num_cores=_NC, num_subcores=_NS)
    return pl.kernel(
        _sc_body,
        out_type=(
            jax.ShapeDtypeStruct((_PAIRS, 8, 128, _D), jnp.float32),
            jax.ShapeDtypeStruct((_PAIRS, 8, 128, _D), jnp.float32),
            jax.ShapeDtypeStruct((_PAIRS, _CACHE), jnp.float32),
        ),
        mesh=mesh,
        scratch_types=[
            pltpu.VMEM((_T,), jnp.float32),
            pltpu.VMEM((16,), jnp.float32),
            pltpu.VMEM((16,), jnp.int32),
            pltpu.VMEM((_CACHE,), jnp.int32),
            pltpu.VMEM((8, 128), jnp.int32),
            pltpu.VMEM((8, 128, _D), jnp.float32),
            pltpu.VMEM((_CACHE,), jnp.float32),
            pltpu.SemaphoreType.DMA,
        ],
        compiler_params=pltpu.CompilerParams(needs_layout_passes=False,
                                             use_tc_tiling_on_sc=False),
    )


def kernel(attn_score_cache, key_cache, value_cache):
    hh, tau_bits, m = pl.pallas_call(
        _tc_body,
        grid=(_B,),
        in_specs=[pl.BlockSpec((1, _H, _Q, _T), lambda b: (b, 0, 0, 0))],
        out_specs=[
            pl.BlockSpec((1, _H, _T), lambda b: (b, 0, 0)),
            pl.BlockSpec((1, _H, 128), lambda b: (b, 0, 0)),
            pl.BlockSpec((1, _H, 128), lambda b: (b, 0, 0)),
        ],
        out_shape=[
            jax.ShapeDtypeStruct((_B, _H, _T), jnp.float32),
            jax.ShapeDtypeStruct((_B, _H, 128), jnp.int32),
            jax.ShapeDtypeStruct((_B, _H, 128), jnp.int32),
        ],
    )(attn_score_cache)

    tau = lax.bitcast_convert_type(tau_bits[:, :, 0], jnp.float32).reshape(-1)
    mm = m[:, :, 0].reshape(-1)
    tau_pad = jnp.pad(tau, (0, 16))
    m_pad = jnp.pad(mm, (0, 16))
    kout, vout, hhout = _make_sc_kernel()(
        hh.reshape(_PAIRS, _T), tau_pad, m_pad,
        key_cache.reshape(_PAIRS * _T, _D), value_cache.reshape(_PAIRS * _T, _D))
    return (kout.reshape(_B, _H, _CACHE, _D),
            vout.reshape(_B, _H, _CACHE, _D),
            hhout.reshape(_B, _H, _CACHE))
